# baseline (speedup 1.0000x reference)
import jax
import jax.numpy as jnp
from jax import lax
from jax.experimental import pallas as pl
from jax.experimental.pallas import tpu as pltpu
from jax.experimental.pallas import tpu_sc as plsc

N_NODES_ = 10000
N_EDGES_ = 320000
D_ = 128
L_ = 16
NC_ = 2
NS_ = 16
NW_ = NC_ * NS_
EDGES_PER_W = N_EDGES_ // NW_
CHUNK = 80
NCHUNKS = EDGES_PER_W // CHUNK

_GATHER_DN = lax.GatherDimensionNumbers(
    offset_dims=(), collapsed_slice_dims=(0,), start_index_map=(0,))

_BITREV4 = [int(f"{k:04b}"[::-1], 2) for k in range(L_)]


def _perm(x, perm):
    return lax.gather(x, perm[:, None], _GATHER_DN, (1,),
                      mode=lax.GatherScatterMode.PROMISE_IN_BOUNDS)


def _sc_body(src_hbm, dst_hbm, h_hbm, out_hbm,
             idx_s, idx_d, u0, v0, u1, v1, o_v,
             sem_u0, sem_v0, sem_u1, sem_v1, sem_o):
    c = lax.axis_index("c")
    s = lax.axis_index("s")
    wid = s * NC_ + c
    base = pl.multiple_of(wid * EDGES_PER_W, EDGES_PER_W)
    lanes = lax.iota(jnp.int32, L_)

    pltpu.sync_copy(src_hbm.at[pl.ds(base, EDGES_PER_W)], idx_s)
    pltpu.sync_copy(dst_hbm.at[pl.ds(base, EDGES_PER_W)], idx_d)

    def issue(ci, ub, vb, su, sv):
        off = pl.multiple_of(ci * CHUNK, CHUNK)
        pltpu.async_copy(h_hbm.at[idx_s.at[pl.ds(off, CHUNK)]], ub, su)
        pltpu.async_copy(h_hbm.at[idx_d.at[pl.ds(off, CHUNK)]], vb, sv)

    def drain(ub, vb, su, sv):
        pltpu.make_async_copy(h_hbm.at[pl.ds(0, CHUNK)], ub, su).wait()
        pltpu.make_async_copy(h_hbm.at[pl.ds(0, CHUNK)], vb, sv).wait()

    p_fold = [
        (lanes + 8) % L_,
        (lanes & 8) | ((lanes + 4) & 7),
        (lanes & 12) | ((lanes + 2) & 3),
        (lanes & 14) | ((lanes + 1) & 1),
    ]
    p_align = [None, (lanes + 12) % L_, (lanes + 14) % L_, (lanes + 15) % L_]
    m_keep = [lanes < 8, (lanes & 4) == 0, (lanes & 2) == 0, (lanes & 1) == 0]

    def compute(ci, ub, vb):
        obase = pl.multiple_of(ci * CHUNK, CHUNK)

        @plsc.parallel_loop(0, CHUNK // L_, unroll=1)
        def group_body(g):
            e0 = g * L_
            vs = []
            for k in range(L_):
                e = e0 + _BITREV4[k]
                prods = [ub[e, pl.ds(j * L_, L_)] * vb[e, pl.ds(j * L_, L_)]
                         for j in range(D_ // L_)]
                while len(prods) > 1:
                    prods = [prods[i] + prods[i + 1]
                             for i in range(0, len(prods), 2)]
                vs.append(prods[0])
            for t in range(4):
                vs = [v + _perm(v, p_fold[t]) for v in vs]
                vs = [jnp.where(m_keep[t], vs[i],
                                vs[i + 1] if p_align[t] is None
                                else _perm(vs[i + 1], p_align[t]))
                      for i in range(0, len(vs), 2)]
            o_v[pl.ds(obase + e0, L_)] = vs[0]

    issue(0, u0, v0, sem_u0, sem_v0)

    def pair_body(g, carry):
        ci0 = 2 * g
        issue(ci0 + 1, u1, v1, sem_u1, sem_v1)
        drain(u0, v0, sem_u0, sem_v0)
        compute(ci0, u0, v0)
        issue(ci0 + 2, u0, v0, sem_u0, sem_v0)
        drain(u1, v1, sem_u1, sem_v1)
        compute(ci0 + 1, u1, v1)
        return carry

    lax.fori_loop(0, (NCHUNKS - 1) // 2, pair_body, 0)
    drain(u0, v0, sem_u0, sem_v0)
    compute(NCHUNKS - 1, u0, v0)

    pltpu.async_copy(o_v, out_hbm.at[pl.ds(base, EDGES_PER_W)], sem_o).wait()


def kernel(h, edge_index):
    src = edge_index[0].astype(jnp.int32)
    dst = edge_index[1].astype(jnp.int32)
    h = h.astype(jnp.float32)

    mesh = plsc.VectorSubcoreMesh(core_axis_name="c", subcore_axis_name="s",
                                  num_cores=NC_, num_subcores=NS_)
    run = pl.kernel(
        _sc_body,
        out_type=jax.ShapeDtypeStruct((N_EDGES_,), jnp.float32),
        mesh=mesh,
        scratch_types=[
            pltpu.VMEM((EDGES_PER_W,), jnp.int32),
            pltpu.VMEM((EDGES_PER_W,), jnp.int32),
            pltpu.VMEM((CHUNK, D_), jnp.float32),
            pltpu.VMEM((CHUNK, D_), jnp.float32),
            pltpu.VMEM((CHUNK, D_), jnp.float32),
            pltpu.VMEM((CHUNK, D_), jnp.float32),
            pltpu.VMEM((EDGES_PER_W,), jnp.float32),
            pltpu.SemaphoreType.DMA,
            pltpu.SemaphoreType.DMA,
            pltpu.SemaphoreType.DMA,
            pltpu.SemaphoreType.DMA,
            pltpu.SemaphoreType.DMA,
        ],
    )
    return run(src, dst, h)


# speedup vs baseline: 2.7081x; 2.7081x over previous
import jax
import jax.numpy as jnp
from jax import lax
from jax.experimental import pallas as pl
from jax.experimental.pallas import tpu as pltpu
from jax.experimental.pallas import tpu_sc as plsc

N_NODES_ = 10000
N_EDGES_ = 320000
D_ = 128
L_ = 16
NC_ = 2
NS_ = 16
NW_ = NC_ * NS_
EDGES_PER_W = N_EDGES_ // NW_
CHUNK = 80
NCHUNKS = EDGES_PER_W // CHUNK

_GATHER_DN = lax.GatherDimensionNumbers(
    offset_dims=(), collapsed_slice_dims=(0,), start_index_map=(0,))

_BITREV4 = [int(f"{k:04b}"[::-1], 2) for k in range(L_)]


def _perm(x, perm):
    return lax.gather(x, perm[:, None], _GATHER_DN, (1,),
                      mode=lax.GatherScatterMode.PROMISE_IN_BOUNDS)


def _sc_body(src_hbm, dst_hbm, h_hbm, out_hbm,
             u0, v0, u1, v1, acc_v, o_v, idx_s, idx_d,
             sem_u0, sem_v0, sem_u1, sem_v1, sem_o):
    c = lax.axis_index("c")
    s = lax.axis_index("s")
    wid = s * NC_ + c
    base = pl.multiple_of(wid * EDGES_PER_W, EDGES_PER_W)
    lanes = lax.iota(jnp.int32, L_)

    pltpu.sync_copy(src_hbm.at[pl.ds(base, EDGES_PER_W)], idx_s)
    pltpu.sync_copy(dst_hbm.at[pl.ds(base, EDGES_PER_W)], idx_d)

    def issue(ci, ub, vb, su, sv):
        off = pl.multiple_of(ci * CHUNK, CHUNK)
        pltpu.async_copy(h_hbm.at[idx_s.at[pl.ds(off, CHUNK)]], ub, su)
        pltpu.async_copy(h_hbm.at[idx_d.at[pl.ds(off, CHUNK)]], vb, sv)

    def drain(ub, vb, su, sv):
        pltpu.make_async_copy(h_hbm.at[pl.ds(0, CHUNK)], ub, su).wait()
        pltpu.make_async_copy(h_hbm.at[pl.ds(0, CHUNK)], vb, sv).wait()

    p_fold = [
        (lanes + 8) % L_,
        (lanes & 8) | ((lanes + 4) & 7),
        (lanes & 12) | ((lanes + 2) & 3),
        (lanes & 14) | ((lanes + 1) & 1),
    ]
    p_align = [None, (lanes + 12) % L_, (lanes + 14) % L_, (lanes + 15) % L_]
    m_keep = [lanes < 8, (lanes & 4) == 0, (lanes & 2) == 0, (lanes & 1) == 0]

    def compute(ci, ub, vb):
        obase = pl.multiple_of(ci * CHUNK, CHUNK)

        @plsc.parallel_loop(0, CHUNK, unroll=4)
        def edge_body(e):
            prods = [ub[e, pl.ds(j * L_, L_)] * vb[e, pl.ds(j * L_, L_)]
                     for j in range(D_ // L_)]
            while len(prods) > 1:
                prods = [prods[i] + prods[i + 1]
                         for i in range(0, len(prods), 2)]
            acc = prods[0]
            acc = acc + _perm(acc, p_fold[0])
            eo = pl.multiple_of(e * L_, L_)
            acc_v[pl.ds(eo, L_)] = acc

        def group_body(g, carry2):
            e0 = g * L_
            vs = []
            for k in range(L_):
                ko = pl.multiple_of((e0 + _BITREV4[k]) * L_, L_)
                vs.append(acc_v[pl.ds(ko, L_)])
            vs = [jnp.where(m_keep[0], vs[i], vs[i + 1])
                  for i in range(0, L_, 2)]
            for t in range(1, 4):
                vs = [v + _perm(v, p_fold[t]) for v in vs]
                vs = [jnp.where(m_keep[t], vs[i], _perm(vs[i + 1], p_align[t]))
                      for i in range(0, len(vs), 2)]
            o_v[pl.ds(obase + e0, L_)] = vs[0]
            return carry2

        lax.fori_loop(0, CHUNK // L_, group_body, 0)

    issue(0, u0, v0, sem_u0, sem_v0)

    def pair_body(g, carry):
        ci0 = 2 * g
        issue(ci0 + 1, u1, v1, sem_u1, sem_v1)
        drain(u0, v0, sem_u0, sem_v0)
        compute(ci0, u0, v0)
        issue(ci0 + 2, u0, v0, sem_u0, sem_v0)
        drain(u1, v1, sem_u1, sem_v1)
        compute(ci0 + 1, u1, v1)
        return carry

    lax.fori_loop(0, (NCHUNKS - 1) // 2, pair_body, 0)
    drain(u0, v0, sem_u0, sem_v0)
    compute(NCHUNKS - 1, u0, v0)

    pltpu.async_copy(o_v, out_hbm.at[pl.ds(base, EDGES_PER_W)], sem_o).wait()


def kernel(h, edge_index):
    src = edge_index[0].astype(jnp.int32)
    dst = edge_index[1].astype(jnp.int32)
    h = h.astype(jnp.float32)

    mesh = plsc.VectorSubcoreMesh(core_axis_name="c", subcore_axis_name="s",
                                  num_cores=NC_, num_subcores=NS_)
    run = pl.kernel(
        _sc_body,
        out_type=jax.ShapeDtypeStruct((N_EDGES_,), jnp.float32),
        mesh=mesh,
        scratch_types=[
            pltpu.VMEM((CHUNK, D_), jnp.float32),
            pltpu.VMEM((CHUNK, D_), jnp.float32),
            pltpu.VMEM((CHUNK, D_), jnp.float32),
            pltpu.VMEM((CHUNK, D_), jnp.float32),
            pltpu.VMEM((CHUNK * L_,), jnp.float32),
            pltpu.VMEM((EDGES_PER_W,), jnp.float32),
            pltpu.VMEM((EDGES_PER_W,), jnp.int32),
            pltpu.VMEM((EDGES_PER_W,), jnp.int32),
            pltpu.SemaphoreType.DMA,
            pltpu.SemaphoreType.DMA,
            pltpu.SemaphoreType.DMA,
            pltpu.SemaphoreType.DMA,
            pltpu.SemaphoreType.DMA,
        ],
    )
    return run(src, dst, h)


# speedup vs baseline: 2.9419x; 1.0863x over previous
import jax
import jax.numpy as jnp
from jax import lax
from jax.experimental import pallas as pl
from jax.experimental.pallas import tpu as pltpu
from jax.experimental.pallas import tpu_sc as plsc

N_NODES_ = 10000
N_EDGES_ = 320000
D_ = 128
L_ = 16
NC_ = 2
NS_ = 16
NW_ = NC_ * NS_
EDGES_PER_W = N_EDGES_ // NW_
CHUNK = 160
STREAM = 80
NFULL = EDGES_PER_W // CHUNK
TAIL = EDGES_PER_W - NFULL * CHUNK

_GATHER_DN = lax.GatherDimensionNumbers(
    offset_dims=(), collapsed_slice_dims=(0,), start_index_map=(0,))

_BITREV4 = [int(f"{k:04b}"[::-1], 2) for k in range(L_)]


def _perm(x, perm):
    return lax.gather(x, perm[:, None], _GATHER_DN, (1,),
                      mode=lax.GatherScatterMode.PROMISE_IN_BOUNDS)


def _sc_body(src_hbm, dst_hbm, h_hbm, out_hbm,
             u0, v0, u1, v1, acc_v, o_v, idx_s, idx_d,
             sem_u0, sem_v0, sem_u1, sem_v1, sem_o):
    c = lax.axis_index("c")
    s = lax.axis_index("s")
    wid = s * NC_ + c
    base = pl.multiple_of(wid * EDGES_PER_W, EDGES_PER_W)
    lanes = lax.iota(jnp.int32, L_)

    pltpu.sync_copy(src_hbm.at[pl.ds(base, EDGES_PER_W)], idx_s)
    pltpu.sync_copy(dst_hbm.at[pl.ds(base, EDGES_PER_W)], idx_d)

    def issue(off, n, ub, vb, su, sv):
        for t in range(n // STREAM):
            so = pl.multiple_of(off + t * STREAM, STREAM)
            pltpu.async_copy(h_hbm.at[idx_s.at[pl.ds(so, STREAM)]],
                             ub.at[pl.ds(t * STREAM, STREAM)], su)
            pltpu.async_copy(h_hbm.at[idx_d.at[pl.ds(so, STREAM)]],
                             vb.at[pl.ds(t * STREAM, STREAM)], sv)

    def drain(n, ub, vb, su, sv):
        pltpu.make_async_copy(h_hbm.at[pl.ds(0, n)],
                              ub.at[pl.ds(0, n)], su).wait()
        pltpu.make_async_copy(h_hbm.at[pl.ds(0, n)],
                              vb.at[pl.ds(0, n)], sv).wait()

    p_fold = [
        (lanes + 8) % L_,
        (lanes & 8) | ((lanes + 4) & 7),
        (lanes & 12) | ((lanes + 2) & 3),
        (lanes & 14) | ((lanes + 1) & 1),
    ]
    p_align = [None, (lanes + 12) % L_, (lanes + 14) % L_, (lanes + 15) % L_]
    m_keep = [lanes < 8, (lanes & 4) == 0, (lanes & 2) == 0, (lanes & 1) == 0]

    def compute(obase, n, ub, vb):
        @plsc.parallel_loop(0, n, unroll=4)
        def edge_body(e):
            prods = [ub[e, pl.ds(j * L_, L_)] * vb[e, pl.ds(j * L_, L_)]
                     for j in range(D_ // L_)]
            while len(prods) > 1:
                prods = [prods[i] + prods[i + 1]
                         for i in range(0, len(prods), 2)]
            acc = prods[0]
            acc = acc + _perm(acc, p_fold[0])
            eo = pl.multiple_of(e * L_, L_)
            acc_v[pl.ds(eo, L_)] = acc

        def group_body(g, carry2):
            e0 = g * L_
            vs = []
            for k in range(L_):
                ko = pl.multiple_of((e0 + _BITREV4[k]) * L_, L_)
                vs.append(acc_v[pl.ds(ko, L_)])
            vs = [jnp.where(m_keep[0], vs[i], vs[i + 1])
                  for i in range(0, L_, 2)]
            for t in range(1, 4):
                vs = [v + _perm(v, p_fold[t]) for v in vs]
                vs = [jnp.where(m_keep[t], vs[i], _perm(vs[i + 1], p_align[t]))
                      for i in range(0, len(vs), 2)]
            o_v[pl.ds(obase + e0, L_)] = vs[0]
            return carry2

        lax.fori_loop(0, n // L_, group_body, 0)

    def coff(ci):
        return pl.multiple_of(ci * CHUNK, CHUNK)

    issue(coff(0), CHUNK, u0, v0, sem_u0, sem_v0)

    def pair_body(g, carry):
        ci0 = 2 * g
        issue(coff(ci0 + 1), CHUNK, u1, v1, sem_u1, sem_v1)
        drain(CHUNK, u0, v0, sem_u0, sem_v0)
        compute(coff(ci0), CHUNK, u0, v0)
        issue(coff(ci0 + 2), CHUNK, u0, v0, sem_u0, sem_v0)
        drain(CHUNK, u1, v1, sem_u1, sem_v1)
        compute(coff(ci0 + 1), CHUNK, u1, v1)
        return carry

    lax.fori_loop(0, (NFULL - 2) // 2, pair_body, 0)
    issue((NFULL - 1) * CHUNK, CHUNK, u1, v1, sem_u1, sem_v1)
    drain(CHUNK, u0, v0, sem_u0, sem_v0)
    compute((NFULL - 2) * CHUNK, CHUNK, u0, v0)
    issue(NFULL * CHUNK, TAIL, u0, v0, sem_u0, sem_v0)
    drain(CHUNK, u1, v1, sem_u1, sem_v1)
    compute((NFULL - 1) * CHUNK, CHUNK, u1, v1)
    drain(TAIL, u0, v0, sem_u0, sem_v0)
    compute(NFULL * CHUNK, TAIL, u0, v0)

    pltpu.async_copy(o_v, out_hbm.at[pl.ds(base, EDGES_PER_W)], sem_o).wait()


def kernel(h, edge_index):
    src = edge_index[0].astype(jnp.int32)
    dst = edge_index[1].astype(jnp.int32)
    h = h.astype(jnp.float32)

    mesh = plsc.VectorSubcoreMesh(core_axis_name="c", subcore_axis_name="s",
                                  num_cores=NC_, num_subcores=NS_)
    run = pl.kernel(
        _sc_body,
        out_type=jax.ShapeDtypeStruct((N_EDGES_,), jnp.float32),
        mesh=mesh,
        scratch_types=[
            pltpu.VMEM((CHUNK, D_), jnp.float32),
            pltpu.VMEM((CHUNK, D_), jnp.float32),
            pltpu.VMEM((CHUNK, D_), jnp.float32),
            pltpu.VMEM((CHUNK, D_), jnp.float32),
            pltpu.VMEM((CHUNK * L_,), jnp.float32),
            pltpu.VMEM((EDGES_PER_W,), jnp.float32),
            pltpu.VMEM((EDGES_PER_W,), jnp.int32),
            pltpu.VMEM((EDGES_PER_W,), jnp.int32),
            pltpu.SemaphoreType.DMA,
            pltpu.SemaphoreType.DMA,
            pltpu.SemaphoreType.DMA,
            pltpu.SemaphoreType.DMA,
            pltpu.SemaphoreType.DMA,
        ],
    )
    return run(src, dst, h)
